# Initial kernel scaffold; baseline (speedup 1.0000x reference)
import jax
import jax.numpy as jnp
from jax.experimental import pallas as pl


def kernel(x, weights, bias, row, col):
    raise NotImplementedError("write your pallas kernel here")


# speedup vs baseline: 8.5895x; 8.5895x over previous
import functools

import jax
import jax.numpy as jnp
from jax import lax
from jax.experimental import pallas as pl
from jax.experimental.pallas import tpu as pltpu
from jax.experimental.pallas import tpu_sc as plsc

N = 10000
B = 64
E = 320000
ITERS = 20
LEAK = 0.01

NC = 2
NS = 16
Bh = B // NC
R = N // NS
K = 128
EperT = -(-E // NS)
NCH = -(-EperT // K)
EP = NS * NCH * K
HL = Bh // 16


def _act(v):
    return jnp.maximum(v, 0.0) + LEAK * jnp.minimum(v, 0.0)


def _sc_body(binc, colp, rowp, wp, out, acc_sh, colv, rowv, wv,
             binv, slab, gbuf, gsem, ssem):
    c = lax.axis_index("c")
    s = lax.axis_index("s")

    pltpu.sync_copy(colp.at[s], colv)
    pltpu.sync_copy(rowp.at[s], rowv)
    pltpu.sync_copy(wp.at[s], wv)
    pltpu.sync_copy(binc.at[c, s], binv)

    def zero_row(r, carry):
        for h in range(HL):
            slab[r, pl.ds(h * 16, 16)] = jnp.zeros((16,), jnp.float32)
        return carry

    def init_row(r, carry):
        for h in range(HL):
            slab[r, pl.ds(h * 16, 16)] = _act(binv[r, pl.ds(h * 16, 16)])
        return carry

    lax.fori_loop(0, R, init_row, 0)
    pltpu.sync_copy(slab, out.at[c, pl.ds(s * R, R)])
    lax.fori_loop(0, R, zero_row, 0)
    pltpu.sync_copy(slab, acc_sh.at[pl.ds(s * R, R)])
    plsc.subcore_barrier()

    def chunk(j, carry):
        pltpu.async_copy(out.at[c].at[colv.at[j]], gbuf, gsem).wait()
        for g in range(K // 16):
            wvec = wv[j, pl.ds(g * 16, 16)]
            for k in range(16):
                bc = jnp.take_along_axis(
                    wvec, jnp.full((16,), k, jnp.int32), axis=0)
                r = g * 16 + k
                for h in range(HL):
                    gbuf[r, pl.ds(h * 16, 16)] = gbuf[r, pl.ds(h * 16, 16)] * bc
        pltpu.async_copy(gbuf, acc_sh.at[rowv.at[j]], ssem, add=True).wait()
        return carry

    def upd_row(r, carry):
        for h in range(HL):
            v = slab[r, pl.ds(h * 16, 16)] + binv[r, pl.ds(h * 16, 16)]
            slab[r, pl.ds(h * 16, 16)] = _act(v)
        return carry

    def iteration(it, carry):
        lax.fori_loop(0, NCH, chunk, 0)
        plsc.subcore_barrier()
        pltpu.sync_copy(acc_sh.at[pl.ds(s * R, R)], slab)
        lax.fori_loop(0, R, upd_row, 0)
        pltpu.sync_copy(slab, out.at[c, pl.ds(s * R, R)])
        lax.fori_loop(0, R, zero_row, 0)
        pltpu.sync_copy(slab, acc_sh.at[pl.ds(s * R, R)])
        plsc.subcore_barrier()
        return carry

    lax.fori_loop(0, ITERS - 1, iteration, 0)


@jax.jit
def _run(binc, colp, rowp, wp):
    f = pl.kernel(
        _sc_body,
        out_type=jax.ShapeDtypeStruct((NC, N, Bh), jnp.float32),
        mesh=plsc.VectorSubcoreMesh(core_axis_name="c", subcore_axis_name="s"),
        compiler_params=pltpu.CompilerParams(use_tc_tiling_on_sc=False),
        scratch_types=[
            pltpu.VMEM_SHARED((N, Bh), jnp.float32),
            pltpu.VMEM((NCH, K), jnp.int32),
            pltpu.VMEM((NCH, K), jnp.int32),
            pltpu.VMEM((NCH, K), jnp.float32),
            pltpu.VMEM((R, Bh), jnp.float32),
            pltpu.VMEM((R, Bh), jnp.float32),
            pltpu.VMEM((K, Bh), jnp.float32),
            pltpu.SemaphoreType.DMA,
            pltpu.SemaphoreType.DMA,
        ],
    )
    return f(binc, colp, rowp, wp)


def kernel(x, weights, bias, row, col):
    row = row.astype(jnp.int32)
    col = col.astype(jnp.int32)
    weights = weights.astype(jnp.float32)
    pad = EP - E
    colp = jnp.concatenate([col, jnp.zeros((pad,), jnp.int32)]).reshape(NS, NCH, K)
    rowp = jnp.concatenate([row, jnp.zeros((pad,), jnp.int32)]).reshape(NS, NCH, K)
    wp = jnp.concatenate([weights, jnp.zeros((pad,), jnp.float32)]).reshape(NS, NCH, K)
    bIn = x.T + bias
    binc = bIn.reshape(N, NC, Bh).transpose(1, 0, 2)
    binc = binc.reshape(NC, NS, R, Bh)
    out = _run(binc, colp, rowp, wp)
    return out.transpose(1, 0, 2).reshape(N, B).T


# speedup vs baseline: 13.8864x; 1.6167x over previous
import functools

import jax
import jax.numpy as jnp
from jax import lax
from jax.experimental import pallas as pl
from jax.experimental.pallas import tpu as pltpu
from jax.experimental.pallas import tpu_sc as plsc

N = 10000
B = 64
E = 320000
ITERS = 20
LEAK = 0.01

NC = 2
NS = 16
Bh = B // NC
R = N // NS
K = 128
EperT = -(-E // NS)
NCH = 2 * (-(-EperT // (2 * K)))
EP = NS * NCH * K
HL = Bh // 16
RC = 125


def _act(v):
    return jnp.maximum(v, 0.0) + LEAK * jnp.minimum(v, 0.0)


def _sc_body(binc, colp, rowp, wp, out, acc_sh, colv, rowv, wv,
             binv, gbuf, sbuf, gsem0, gsem1, ssem0, ssem1):
    c = lax.axis_index("c")
    s = lax.axis_index("s")
    gsem = (gsem0, gsem1)
    ssem = (ssem0, ssem1)

    pltpu.sync_copy(colp.at[s], colv)
    pltpu.sync_copy(rowp.at[s], rowv)
    pltpu.sync_copy(wp.at[s], wv)
    pltpu.sync_copy(binc.at[c, s], binv)

    for t in range(R // RC):
        def init_row(r, carry, t=t):
            for h in range(HL):
                gbuf[0, r, pl.ds(h * 16, 16)] = _act(
                    binv[t * RC + r, pl.ds(h * 16, 16)])
            return carry
        lax.fori_loop(0, RC, init_row, 0)
        pltpu.sync_copy(gbuf.at[0, pl.ds(0, RC)],
                        out.at[c, pl.ds(s * R + t * RC, RC)])
    pltpu.sync_copy(binv, acc_sh.at[pl.ds(s * R, R)])
    plsc.subcore_barrier()

    def scale(q, b):
        for g in range(K // 16):
            wvec = wv[q, pl.ds(g * 16, 16)]
            for k in range(16):
                bc = jnp.take_along_axis(
                    wvec, jnp.full((16,), k, jnp.int32), axis=0)
                r = g * 16 + k
                for h in range(HL):
                    sbuf[b, r, pl.ds(h * 16, 16)] = (
                        gbuf[b, r, pl.ds(h * 16, 16)] * bc)

    def start_gather(q, b):
        return pltpu.async_copy(out.at[c].at[colv.at[q]], gbuf.at[b],
                                gsem[b])

    def wait_gather(q, b):
        pltpu.make_async_copy(out.at[c].at[colv.at[q]], gbuf.at[b],
                              gsem[b]).wait()

    def start_scatter(q, b):
        return pltpu.async_copy(sbuf.at[b], acc_sh.at[rowv.at[q]],
                                ssem[b], add=True)

    def wait_scatter(q, b):
        pltpu.make_async_copy(sbuf.at[b], acc_sh.at[rowv.at[q]],
                              ssem[b]).wait()

    def iteration(it, carry):
        start_gather(0, 0)
        start_gather(1, 1)

        def pair(i, carry2):
            for b in range(2):
                q = i * 2 + b
                wait_gather(q, b)

                @pl.when(i > 0)
                def _():
                    wait_scatter(q - 2, b)

                scale(q, b)

                @pl.when(q + 2 < NCH)
                def _():
                    start_gather(q + 2, b)

                start_scatter(q, b)
            return carry2

        lax.fori_loop(0, NCH // 2, pair, 0)
        wait_scatter(NCH - 2, 0)
        wait_scatter(NCH - 1, 1)
        plsc.subcore_barrier()

        for t in range(R // RC):
            sl = pl.ds(s * R + t * RC, RC)
            pltpu.sync_copy(acc_sh.at[sl], gbuf.at[0, pl.ds(0, RC)])

            def upd_row(r, carry3):
                for h in range(HL):
                    gbuf[0, r, pl.ds(h * 16, 16)] = _act(
                        gbuf[0, r, pl.ds(h * 16, 16)])
                return carry3

            lax.fori_loop(0, RC, upd_row, 0)
            pltpu.sync_copy(gbuf.at[0, pl.ds(0, RC)], out.at[c, sl])
            pltpu.sync_copy(binv.at[pl.ds(t * RC, RC)], acc_sh.at[sl])
        plsc.subcore_barrier()
        return carry

    lax.fori_loop(0, ITERS - 1, iteration, 0)


@jax.jit
def _run(binc, colp, rowp, wp):
    f = pl.kernel(
        _sc_body,
        out_type=jax.ShapeDtypeStruct((NC, N, Bh), jnp.float32),
        mesh=plsc.VectorSubcoreMesh(core_axis_name="c", subcore_axis_name="s"),
        compiler_params=pltpu.CompilerParams(use_tc_tiling_on_sc=False),
        scratch_types=[
            pltpu.VMEM_SHARED((N, Bh), jnp.float32),
            pltpu.VMEM((NCH, K), jnp.int32),
            pltpu.VMEM((NCH, K), jnp.int32),
            pltpu.VMEM((NCH, K), jnp.float32),
            pltpu.VMEM((R, Bh), jnp.float32),
            pltpu.VMEM((2, K, Bh), jnp.float32),
            pltpu.VMEM((2, K, Bh), jnp.float32),
            pltpu.SemaphoreType.DMA,
            pltpu.SemaphoreType.DMA,
            pltpu.SemaphoreType.DMA,
            pltpu.SemaphoreType.DMA,
        ],
    )
    return f(binc, colp, rowp, wp)


def kernel(x, weights, bias, row, col):
    row = row.astype(jnp.int32)
    col = col.astype(jnp.int32)
    weights = weights.astype(jnp.float32)
    pad = EP - E
    colp = jnp.concatenate([col, jnp.zeros((pad,), jnp.int32)]).reshape(NS, NCH, K)
    rowp = jnp.concatenate([row, jnp.zeros((pad,), jnp.int32)]).reshape(NS, NCH, K)
    wp = jnp.concatenate([weights, jnp.zeros((pad,), jnp.float32)]).reshape(NS, NCH, K)
    bIn = x.T + bias
    binc = bIn.reshape(N, NC, Bh).transpose(1, 0, 2)
    binc = binc.reshape(NC, NS, R, Bh)
    out = _run(binc, colp, rowp, wp)
    return out.transpose(1, 0, 2).reshape(N, B).T
